# baseline (speedup 1.0000x reference)
import functools

import jax
import jax.numpy as jnp
from jax import lax
from jax.experimental import pallas as pl
from jax.experimental.pallas import tpu as pltpu
from jax.experimental.pallas import tpu_sc as plsc

NUM_POIS = 1000000
EMBED_DIM = 64
BATCH = 16384
HIST = 20

_B = BATCH * HIST
_NW = 32
_G = 128
_PER_W = _B // _NW
_NG = _PER_W // _G


def _make_gather():
    mesh = plsc.VectorSubcoreMesh(core_axis_name="c", subcore_axis_name="s")

    @functools.partial(
        pl.kernel,
        mesh=mesh,
        out_type=jax.ShapeDtypeStruct((_B, EMBED_DIM), jnp.float32),
        compiler_params=pltpu.CompilerParams(use_tc_tiling_on_sc=False),
        scratch_types=[
            pltpu.VMEM((_NG, _G), jnp.int32),
            pltpu.VMEM((_G, EMBED_DIM), jnp.float32),
            pltpu.SemaphoreType.DMA,
        ],
    )
    def gather_kernel(table_hbm, idx_hbm, out_hbm, idx_v, rows_v, sem):
        wid = lax.axis_index("s") * 2 + lax.axis_index("c")
        base = wid * _PER_W
        pltpu.sync_copy(idx_hbm.at[pl.ds(wid * _NG, _NG)], idx_v)

        def body(g, carry):
            pltpu.async_copy(table_hbm.at[idx_v.at[g]], rows_v, sem).wait()
            pltpu.sync_copy(rows_v, out_hbm.at[pl.ds(base + g * _G, _G)])
            return carry

        lax.fori_loop(0, _NG, body, 0)

    return gather_kernel


_gather = _make_gather()


def kernel(poi_ids, table):
    idx = poi_ids.reshape(_B // _G, _G).astype(jnp.int32)
    out = _gather(table, idx)
    return out.reshape(BATCH, HIST, EMBED_DIM)


# speedup vs baseline: 1.0597x; 1.0597x over previous
import functools

import jax
import jax.numpy as jnp
from jax import lax
from jax.experimental import pallas as pl
from jax.experimental.pallas import tpu as pltpu
from jax.experimental.pallas import tpu_sc as plsc

NUM_POIS = 1000000
EMBED_DIM = 64
BATCH = 16384
HIST = 20

_B = BATCH * HIST
_NW = 32
_G = 128
_PER_W = _B // _NW
_NG = _PER_W // _G
_NB = 8


def _make_gather():
    mesh = plsc.VectorSubcoreMesh(core_axis_name="c", subcore_axis_name="s")

    @functools.partial(
        pl.kernel,
        mesh=mesh,
        out_type=jax.ShapeDtypeStruct((_B, EMBED_DIM), jnp.float32),
        compiler_params=pltpu.CompilerParams(use_tc_tiling_on_sc=False),
        scratch_types=(
            [pltpu.VMEM((_NG, _G), jnp.int32),
             pltpu.VMEM((_NB, _G, EMBED_DIM), jnp.float32)]
            + [pltpu.SemaphoreType.DMA] * (2 * _NB)
        ),
    )
    def gather_kernel(table_hbm, idx_hbm, out_hbm, idx_v, rows_v, *sems):
        gsem = sems[:_NB]
        ssem = sems[_NB:]
        wid = lax.axis_index("s") * 2 + lax.axis_index("c")
        base = wid * _PER_W
        pltpu.sync_copy(idx_hbm.at[pl.ds(wid * _NG, _NG)], idx_v)

        def gather_grp(g, b):
            pltpu.async_copy(table_hbm.at[idx_v.at[g]], rows_v.at[b], gsem[b])

        def store_grp(g, b):
            pltpu.async_copy(
                rows_v.at[b], out_hbm.at[pl.ds(base + g * _G, _G)], ssem[b])

        for b in range(_NB):
            gather_grp(b, b)

        @pl.loop(0, _NG, step=_NB)
        def outer(t):
            for b in range(_NB):
                g = t + b
                pltpu.make_async_copy(
                    table_hbm.at[idx_v.at[g]], rows_v.at[b], gsem[b]).wait()
                store_grp(g, b)
            for b in range(_NB):
                g2 = t + _NB + b

                @pl.when(g2 < _NG)
                def _():
                    pltpu.make_async_copy(
                        rows_v.at[b],
                        out_hbm.at[pl.ds(base + (t + b) * _G, _G)],
                        ssem[b]).wait()
                    gather_grp(g2, b)

        for b in range(_NB):
            pltpu.make_async_copy(
                rows_v.at[b],
                out_hbm.at[pl.ds(base + (_NG - _NB + b) * _G, _G)],
                ssem[b]).wait()

    return gather_kernel


_gather = _make_gather()


def kernel(poi_ids, table):
    idx = poi_ids.reshape(_B // _G, _G).astype(jnp.int32)
    out = _gather(table, idx)
    return out.reshape(BATCH, HIST, EMBED_DIM)
